# baseline (speedup 1.0000x reference)
import functools
import jax
import jax.numpy as jnp
from jax import lax
from jax.experimental import pallas as pl
from jax.experimental.pallas import tpu as pltpu
from jax.experimental.pallas import tpu_sc as plsc

N_NODES = 10000
N_EDGES = 320000
H = 128
DE = 16
EQW = 16
TAILW = 16
OUTW = H + TAILW

BN = 1000
BE = 1000


def _silu(x):
    return x * jax.nn.sigmoid(x)


def _node_pre_body(h_ref, Wr_ref, Wc_ref, Wg1_ref, bg1_ref, Wg2_ref, bg2_ref,
                   A_ref, B_ref, gate_ref):
    h = h_ref[...]
    A_ref[...] = jnp.dot(h, Wr_ref[...], preferred_element_type=jnp.float32)
    B_ref[...] = jnp.dot(h, Wc_ref[...], preferred_element_type=jnp.float32)
    g1 = _silu(jnp.dot(h, Wg1_ref[...], preferred_element_type=jnp.float32)
               + bg1_ref[...])
    gate_ref[...] = (jnp.dot(g1, Wg2_ref[...], preferred_element_type=jnp.float32)
                     + bg2_ref[...])


def _node_pre(h, p):
    Wr = p['edge']['W1'][1:1 + H]
    Wc = p['edge']['W1'][1 + H:1 + 2 * H]
    Wg1 = p['node_equ']['W1']
    bg1 = p['node_equ']['b1'][None, :]
    Wg2 = p['node_equ']['W2']
    bg2 = p['node_equ']['b2'][None, :]
    grid = (N_NODES // BN,)
    full = lambda r, c: pl.BlockSpec((r, c), lambda i: (0, 0))
    blk = lambda c: pl.BlockSpec((BN, c), lambda i: (i, 0))
    return pl.pallas_call(
        _node_pre_body,
        grid=grid,
        in_specs=[blk(H), full(H, H), full(H, H), full(H, H), full(1, H),
                  full(H, 1), full(1, 1)],
        out_specs=[blk(H), blk(H), blk(1)],
        out_shape=[jax.ShapeDtypeStruct((N_NODES, H), jnp.float32),
                   jax.ShapeDtypeStruct((N_NODES, H), jnp.float32),
                   jax.ShapeDtypeStruct((N_NODES, 1), jnp.float32)],
    )(h, Wr, Wc, Wg1, bg1, Wg2, bg2)


def _embed_body(h_ref, We_ref, be_ref, out_ref):
    out_ref[...] = (jnp.dot(h_ref[...], We_ref[...],
                            preferred_element_type=jnp.float32) + be_ref[...])


def _embed(h, W_emb, b_emb):
    grid = (N_NODES // BN,)
    return pl.pallas_call(
        _embed_body,
        grid=grid,
        in_specs=[pl.BlockSpec((BN, H), lambda i: (i, 0)),
                  pl.BlockSpec((H, H), lambda i: (0, 0)),
                  pl.BlockSpec((1, H), lambda i: (0, 0))],
        out_specs=pl.BlockSpec((BN, H), lambda i: (i, 0)),
        out_shape=jax.ShapeDtypeStruct((N_NODES, H), jnp.float32),
    )(h, W_emb, b_emb[None, :])


def _edge_body(pre1_ref, rij_ref, ef_ref, w1s_ref, W1e_ref, b1_ref,
               W2_ref, b2_ref, Wc1_ref, bc1_ref, Wc2_ref, bc2_ref, out_ref):
    rij = rij_ref[...]
    s2 = jnp.sum(rij * rij, axis=1, keepdims=True) + 1e-12
    sij = jnp.sqrt(s2)
    z = (pre1_ref[...]
         + jnp.dot(ef_ref[...], W1e_ref[...], preferred_element_type=jnp.float32)
         + sij * w1s_ref[...] + b1_ref[...])
    u = _silu(z)
    m = _silu(jnp.dot(u, W2_ref[...], preferred_element_type=jnp.float32)
              + b2_ref[...])
    v = _silu(jnp.dot(m, Wc1_ref[...], preferred_element_type=jnp.float32)
              + bc1_ref[...])
    cm = (jnp.dot(v, Wc2_ref[...], preferred_element_type=jnp.float32)
          + bc2_ref[...])
    ones_col = (jax.lax.broadcasted_iota(jnp.int32, (1, TAILW), 1) == 3
                ).astype(jnp.float32)
    tail = rij * cm + ones_col
    out_ref[...] = jnp.concatenate([m, tail], axis=1)


def _edge_mlp(pre1, rij, edge_fea, p):
    w1s = p['edge']['W1'][0:1]
    W1e = p['edge']['W1'][1 + 2 * H:]
    b1 = p['edge']['b1'][None, :]
    W2 = p['edge']['W2']
    b2 = p['edge']['b2'][None, :]
    Wc1 = p['coord']['W1']
    bc1 = p['coord']['b1'][None, :]
    Wc2 = p['coord']['W2']
    bc2 = p['coord']['b2'][None, :]
    grid = (N_EDGES // BE,)
    full = lambda r, c: pl.BlockSpec((r, c), lambda i: (0, 0))
    blk = lambda c: pl.BlockSpec((BE, c), lambda i: (i, 0))
    return pl.pallas_call(
        _edge_body,
        grid=grid,
        in_specs=[blk(H), blk(EQW), blk(DE), full(1, H), full(DE, H),
                  full(1, H), full(H, H), full(1, H), full(H, H), full(1, H),
                  full(H, 1), full(1, 1)],
        out_specs=blk(OUTW),
        out_shape=jax.ShapeDtypeStruct((N_EDGES, OUTW), jnp.float32),
    )(pre1, rij, edge_fea, w1s, W1e, b1, W2, b2, Wc1, bc1, Wc2, bc2)


def _node_upd_body(h_ref, equ_ref, gate_ref, agg_ref,
                   wequ_ref, Wn1h_ref, Wn1m_ref, bn1_ref, Wn2_ref, bn2_ref,
                   h_out_ref, equ_out_ref):
    agg = jnp.sum(agg_ref[...], axis=0)
    msum = agg[:, :H]
    tail = agg[:, H:]
    cnt = tail[:, 3:4]
    fmean = tail / jnp.maximum(cnt, 1.0)
    mask = (jax.lax.broadcasted_iota(jnp.int32, (1, EQW), 1) < 3
            ).astype(jnp.float32)
    equ_out_ref[...] = (equ_ref[...] * wequ_ref[0, 0] * gate_ref[...]
                        + fmean) * mask
    h = h_ref[...]
    u = _silu(jnp.dot(h, Wn1h_ref[...], preferred_element_type=jnp.float32)
              + jnp.dot(msum, Wn1m_ref[...], preferred_element_type=jnp.float32)
              + bn1_ref[...])
    h_out_ref[...] = (jnp.dot(u, Wn2_ref[...], preferred_element_type=jnp.float32)
                      + bn2_ref[...] + h)


def _node_update(h, equ, gate, agg_slabs, p):
    Wn1h = p['node']['W1'][:H]
    Wn1m = p['node']['W1'][H:]
    bn1 = p['node']['b1'][None, :]
    Wn2 = p['node']['W2']
    bn2 = p['node']['b2'][None, :]
    S = agg_slabs.shape[0]
    grid = (N_NODES // BN,)
    full = lambda r, c: pl.BlockSpec((r, c), lambda i: (0, 0))
    blk = lambda c: pl.BlockSpec((BN, c), lambda i: (i, 0))
    return pl.pallas_call(
        _node_upd_body,
        grid=grid,
        in_specs=[blk(H), blk(EQW), blk(1),
                  pl.BlockSpec((S, BN, OUTW), lambda i: (0, i, 0)),
                  full(1, 1), full(H, H), full(H, H), full(1, H), full(H, H),
                  full(1, H)],
        out_specs=[blk(H), blk(EQW)],
        out_shape=[jax.ShapeDtypeStruct((N_NODES, H), jnp.float32),
                   jax.ShapeDtypeStruct((N_NODES, EQW), jnp.float32)],
    )(h, equ, gate, agg_slabs, p['W_equ'], Wn1h, Wn1m, bn1, Wn2, bn2)


NW = 32
CH = 128
NCHUNKS = N_EDGES // CH
NPS = N_NODES // 16


def _sc_gather(A, B, equ, row, col):
    mesh = plsc.VectorSubcoreMesh(core_axis_name="c", subcore_axis_name="s")

    @functools.partial(
        pl.kernel, mesh=mesh,
        compiler_params=pltpu.CompilerParams(use_tc_tiling_on_sc=False),
        out_type=[jax.ShapeDtypeStruct((N_EDGES, H), jnp.float32),
                  jax.ShapeDtypeStruct((N_EDGES, EQW), jnp.float32)],
        scratch_types=[pltpu.VMEM((CH,), jnp.int32),
                       pltpu.VMEM((CH,), jnp.int32),
                       pltpu.VMEM((CH, H), jnp.float32),
                       pltpu.VMEM((CH, H), jnp.float32),
                       pltpu.VMEM((CH, EQW), jnp.float32),
                       pltpu.VMEM((CH, EQW), jnp.float32),
                       pltpu.VMEM((CH, EQW), jnp.float32)],
    )
    def k(A_h, B_h, equ_h, row_h, col_h, pre1_h, rij_h,
          rv, cv, av, bv, e1, e2, rij_v):
        c = lax.axis_index("c")
        s = lax.axis_index("s")
        wid = s * 2 + c

        @pl.loop(wid, NCHUNKS, step=NW)
        def _(chk):
            base = chk * CH
            pltpu.sync_copy(row_h.at[pl.ds(base, CH)], rv)
            pltpu.sync_copy(col_h.at[pl.ds(base, CH)], cv)
            pltpu.sync_copy(A_h.at[rv], av)
            pltpu.sync_copy(B_h.at[cv], bv)
            pltpu.sync_copy(equ_h.at[rv], e1)
            pltpu.sync_copy(equ_h.at[cv], e2)

            @pl.loop(0, CH)
            def _(i):
                @pl.loop(0, H, step=16)
                def _(j):
                    slc = (i, pl.ds(j, 16))
                    av.at[*slc][...] = av.at[*slc][...] + bv.at[*slc][...]
                eslc = (i, pl.ds(0, 16))
                rij_v.at[i][...] = e1.at[*eslc][...] - e2.at[*eslc][...]

            pltpu.sync_copy(av, pre1_h.at[pl.ds(base, CH)])
            pltpu.sync_copy(rij_v, rij_h.at[pl.ds(base, CH)])

    return k(A, B, equ, row, col)


def _sc_scatter(edge_out, row):
    mesh = plsc.VectorSubcoreMesh(core_axis_name="c", subcore_axis_name="s")

    @functools.partial(
        pl.kernel, mesh=mesh,
        compiler_params=pltpu.CompilerParams(use_tc_tiling_on_sc=False),
        out_type=jax.ShapeDtypeStruct((2, N_NODES, OUTW), jnp.float32),
        scratch_types=[pltpu.VMEM((CH,), jnp.int32),
                       pltpu.VMEM((CH, OUTW), jnp.float32),
                       pltpu.VMEM_SHARED((N_NODES, OUTW), jnp.float32)],
    )
    def k(eo_h, row_h, out_h, idxv, rows, acc):
        c = lax.axis_index("c")
        s = lax.axis_index("s")
        wid = s * 2 + c

        @pl.loop(0, 125)
        def _(i):
            @pl.loop(0, OUTW, step=16)
            def _(j):
                rows.at[i, pl.ds(j, 16)][...] = jnp.zeros((16,), jnp.float32)

        @pl.loop(0, NPS, step=125)
        def _(r):
            pltpu.sync_copy(rows.at[pl.ds(0, 125)],
                            acc.at[pl.ds(s * NPS + r, 125)])

        plsc.subcore_barrier()

        @pl.loop(wid, NCHUNKS, step=NW)
        def _(chk):
            base = chk * CH
            pltpu.sync_copy(row_h.at[pl.ds(base, CH)], idxv)
            pltpu.sync_copy(eo_h.at[pl.ds(base, CH)], rows)
            pltpu.sync_copy(rows, acc.at[idxv], add=True)

        plsc.subcore_barrier()
        pltpu.sync_copy(acc.at[pl.ds(s * NPS, NPS)],
                        out_h.at[c].at[pl.ds(s * NPS, NPS)])

    return k(edge_out, row)


def kernel(x, h, edge_index, edge_fea, W_emb, b_emb, params):
    row = edge_index[0].astype(jnp.int32)
    col = edge_index[1].astype(jnp.int32)
    h = _embed(h, W_emb, b_emb)
    equ = jnp.pad(x, ((0, 0), (0, EQW - 3)))
    for p in params:
        A, B, gate = _node_pre(h, p)
        pre1, rij = _sc_gather(A, B, equ, row, col)
        edge_out = _edge_mlp(pre1, rij, edge_fea, p)
        agg = _sc_scatter(edge_out, row)
        h, equ = _node_update(h, equ, gate, agg, p)
    return equ[:, :3]


# speedup vs baseline: 1.0143x; 1.0143x over previous
import functools
import jax
import jax.numpy as jnp
from jax import lax
from jax.experimental import pallas as pl
from jax.experimental.pallas import tpu as pltpu
from jax.experimental.pallas import tpu_sc as plsc

N_NODES = 10000
N_EDGES = 320000
H = 128
DE = 16
EQW = 16
TAILW = 16
OUTW = H + TAILW

BN = 1000
BE = 1000


def _silu(x):
    return x * jax.nn.sigmoid(x)


def _node_pre_body(h_ref, Wr_ref, Wc_ref, Wg1_ref, bg1_ref, Wg2_ref, bg2_ref,
                   A_ref, B_ref, gate_ref):
    h = h_ref[...]
    A_ref[...] = jnp.dot(h, Wr_ref[...], preferred_element_type=jnp.float32)
    B_ref[...] = jnp.dot(h, Wc_ref[...], preferred_element_type=jnp.float32)
    g1 = _silu(jnp.dot(h, Wg1_ref[...], preferred_element_type=jnp.float32)
               + bg1_ref[...])
    gate_ref[...] = (jnp.dot(g1, Wg2_ref[...], preferred_element_type=jnp.float32)
                     + bg2_ref[...])


def _node_pre(h, p):
    Wr = p['edge']['W1'][1:1 + H]
    Wc = p['edge']['W1'][1 + H:1 + 2 * H]
    Wg1 = p['node_equ']['W1']
    bg1 = p['node_equ']['b1'][None, :]
    Wg2 = p['node_equ']['W2']
    bg2 = p['node_equ']['b2'][None, :]
    grid = (N_NODES // BN,)
    full = lambda r, c: pl.BlockSpec((r, c), lambda i: (0, 0))
    blk = lambda c: pl.BlockSpec((BN, c), lambda i: (i, 0))
    return pl.pallas_call(
        _node_pre_body,
        grid=grid,
        in_specs=[blk(H), full(H, H), full(H, H), full(H, H), full(1, H),
                  full(H, 1), full(1, 1)],
        out_specs=[blk(H), blk(H), blk(1)],
        out_shape=[jax.ShapeDtypeStruct((N_NODES, H), jnp.float32),
                   jax.ShapeDtypeStruct((N_NODES, H), jnp.float32),
                   jax.ShapeDtypeStruct((N_NODES, 1), jnp.float32)],
    )(h, Wr, Wc, Wg1, bg1, Wg2, bg2)


def _embed_body(h_ref, We_ref, be_ref, out_ref):
    out_ref[...] = (jnp.dot(h_ref[...], We_ref[...],
                            preferred_element_type=jnp.float32) + be_ref[...])


def _embed(h, W_emb, b_emb):
    grid = (N_NODES // BN,)
    return pl.pallas_call(
        _embed_body,
        grid=grid,
        in_specs=[pl.BlockSpec((BN, H), lambda i: (i, 0)),
                  pl.BlockSpec((H, H), lambda i: (0, 0)),
                  pl.BlockSpec((1, H), lambda i: (0, 0))],
        out_specs=pl.BlockSpec((BN, H), lambda i: (i, 0)),
        out_shape=jax.ShapeDtypeStruct((N_NODES, H), jnp.float32),
    )(h, W_emb, b_emb[None, :])


def _edge_body(pre1_ref, rij_ref, ef_ref, w1s_ref, W1e_ref, b1_ref,
               W2_ref, b2_ref, Wc1_ref, bc1_ref, Wc2_ref, bc2_ref, out_ref):
    rij = rij_ref[...]
    s2 = jnp.sum(rij * rij, axis=1, keepdims=True) + 1e-12
    sij = jnp.sqrt(s2)
    z = (pre1_ref[...]
         + jnp.dot(ef_ref[...], W1e_ref[...], preferred_element_type=jnp.float32)
         + sij * w1s_ref[...] + b1_ref[...])
    u = _silu(z)
    m = _silu(jnp.dot(u, W2_ref[...], preferred_element_type=jnp.float32)
              + b2_ref[...])
    v = _silu(jnp.dot(m, Wc1_ref[...], preferred_element_type=jnp.float32)
              + bc1_ref[...])
    cm = (jnp.dot(v, Wc2_ref[...], preferred_element_type=jnp.float32)
          + bc2_ref[...])
    ones_col = (jax.lax.broadcasted_iota(jnp.int32, (1, TAILW), 1) == 3
                ).astype(jnp.float32)
    tail = rij * cm + ones_col
    out_ref[...] = jnp.concatenate([m, tail], axis=1)


def _edge_mlp(pre1, rij, edge_fea, p):
    w1s = p['edge']['W1'][0:1]
    W1e = p['edge']['W1'][1 + 2 * H:]
    b1 = p['edge']['b1'][None, :]
    W2 = p['edge']['W2']
    b2 = p['edge']['b2'][None, :]
    Wc1 = p['coord']['W1']
    bc1 = p['coord']['b1'][None, :]
    Wc2 = p['coord']['W2']
    bc2 = p['coord']['b2'][None, :]
    grid = (N_EDGES // BE,)
    full = lambda r, c: pl.BlockSpec((r, c), lambda i: (0, 0))
    blk = lambda c: pl.BlockSpec((BE, c), lambda i: (i, 0))
    return pl.pallas_call(
        _edge_body,
        grid=grid,
        in_specs=[blk(H), blk(EQW), blk(DE), full(1, H), full(DE, H),
                  full(1, H), full(H, H), full(1, H), full(H, H), full(1, H),
                  full(H, 1), full(1, 1)],
        out_specs=blk(OUTW),
        out_shape=jax.ShapeDtypeStruct((N_EDGES, OUTW), jnp.float32),
    )(pre1, rij, edge_fea, w1s, W1e, b1, W2, b2, Wc1, bc1, Wc2, bc2)


def _node_upd_body(h_ref, equ_ref, gate_ref, agg_ref,
                   wequ_ref, Wn1h_ref, Wn1m_ref, bn1_ref, Wn2_ref, bn2_ref,
                   h_out_ref, equ_out_ref):
    agg = jnp.sum(agg_ref[...], axis=0)
    msum = agg[:, :H]
    tail = agg[:, H:]
    cnt = tail[:, 3:4]
    fmean = tail / jnp.maximum(cnt, 1.0)
    mask = (jax.lax.broadcasted_iota(jnp.int32, (1, EQW), 1) < 3
            ).astype(jnp.float32)
    equ_out_ref[...] = (equ_ref[...] * wequ_ref[0, 0] * gate_ref[...]
                        + fmean) * mask
    h = h_ref[...]
    u = _silu(jnp.dot(h, Wn1h_ref[...], preferred_element_type=jnp.float32)
              + jnp.dot(msum, Wn1m_ref[...], preferred_element_type=jnp.float32)
              + bn1_ref[...])
    h_out_ref[...] = (jnp.dot(u, Wn2_ref[...], preferred_element_type=jnp.float32)
                      + bn2_ref[...] + h)


def _node_update(h, equ, gate, agg_slabs, p):
    Wn1h = p['node']['W1'][:H]
    Wn1m = p['node']['W1'][H:]
    bn1 = p['node']['b1'][None, :]
    Wn2 = p['node']['W2']
    bn2 = p['node']['b2'][None, :]
    S = agg_slabs.shape[0]
    grid = (N_NODES // BN,)
    full = lambda r, c: pl.BlockSpec((r, c), lambda i: (0, 0))
    blk = lambda c: pl.BlockSpec((BN, c), lambda i: (i, 0))
    return pl.pallas_call(
        _node_upd_body,
        grid=grid,
        in_specs=[blk(H), blk(EQW), blk(1),
                  pl.BlockSpec((S, BN, OUTW), lambda i: (0, i, 0)),
                  full(1, 1), full(H, H), full(H, H), full(1, H), full(H, H),
                  full(1, H)],
        out_specs=[blk(H), blk(EQW)],
        out_shape=[jax.ShapeDtypeStruct((N_NODES, H), jnp.float32),
                   jax.ShapeDtypeStruct((N_NODES, EQW), jnp.float32)],
    )(h, equ, gate, agg_slabs, p['W_equ'], Wn1h, Wn1m, bn1, Wn2, bn2)


NW = 32
CH = 128
NCHUNKS = N_EDGES // CH
NPS = N_NODES // 16


AW = H + EQW


def _sc_gather(A144, B144, row, col):
    mesh = plsc.VectorSubcoreMesh(core_axis_name="c", subcore_axis_name="s")

    @functools.partial(
        pl.kernel, mesh=mesh,
        compiler_params=pltpu.CompilerParams(use_tc_tiling_on_sc=False),
        out_type=[jax.ShapeDtypeStruct((N_EDGES, H), jnp.float32),
                  jax.ShapeDtypeStruct((N_EDGES, EQW), jnp.float32)],
        scratch_types=[pltpu.VMEM((CH,), jnp.int32),
                       pltpu.VMEM((CH,), jnp.int32),
                       pltpu.VMEM((CH, AW), jnp.float32),
                       pltpu.VMEM((CH, AW), jnp.float32),
                       pltpu.VMEM((CH, H), jnp.float32),
                       pltpu.VMEM((CH, EQW), jnp.float32)],
    )
    def k(A_h, B_h, row_h, col_h, pre1_h, rij_h,
          rv, cv, av, bv, pre1_v, rij_v):
        c = lax.axis_index("c")
        s = lax.axis_index("s")
        wid = s * 2 + c

        @pl.loop(wid, NCHUNKS, step=NW)
        def _(chk):
            base = chk * CH
            pltpu.sync_copy(row_h.at[pl.ds(base, CH)], rv)
            pltpu.sync_copy(col_h.at[pl.ds(base, CH)], cv)
            pltpu.sync_copy(A_h.at[rv], av)
            pltpu.sync_copy(B_h.at[cv], bv)

            @pl.loop(0, CH)
            def _(i):
                for j in range(0, H, 16):
                    pre1_v.at[i, pl.ds(j, 16)][...] = (
                        av.at[i, pl.ds(j, 16)][...]
                        + bv.at[i, pl.ds(j, 16)][...])
                rij_v.at[i][...] = (av.at[i, pl.ds(H, EQW)][...]
                                    - bv.at[i, pl.ds(H, EQW)][...])

            pltpu.sync_copy(pre1_v, pre1_h.at[pl.ds(base, CH)])
            pltpu.sync_copy(rij_v, rij_h.at[pl.ds(base, CH)])

    return k(A144, B144, row, col)


def _sc_scatter(edge_out, row):
    mesh = plsc.VectorSubcoreMesh(core_axis_name="c", subcore_axis_name="s")

    @functools.partial(
        pl.kernel, mesh=mesh,
        compiler_params=pltpu.CompilerParams(use_tc_tiling_on_sc=False),
        out_type=jax.ShapeDtypeStruct((2, N_NODES, OUTW), jnp.float32),
        scratch_types=[pltpu.VMEM((CH,), jnp.int32),
                       pltpu.VMEM((CH, OUTW), jnp.float32),
                       pltpu.VMEM_SHARED((N_NODES, OUTW), jnp.float32)],
    )
    def k(eo_h, row_h, out_h, idxv, rows, acc):
        c = lax.axis_index("c")
        s = lax.axis_index("s")
        wid = s * 2 + c

        @pl.loop(0, 125)
        def _(i):
            @pl.loop(0, OUTW, step=16)
            def _(j):
                rows.at[i, pl.ds(j, 16)][...] = jnp.zeros((16,), jnp.float32)

        @pl.loop(0, NPS, step=125)
        def _(r):
            pltpu.sync_copy(rows.at[pl.ds(0, 125)],
                            acc.at[pl.ds(s * NPS + r, 125)])

        plsc.subcore_barrier()

        @pl.loop(wid, NCHUNKS, step=NW)
        def _(chk):
            base = chk * CH
            pltpu.sync_copy(row_h.at[pl.ds(base, CH)], idxv)
            pltpu.sync_copy(eo_h.at[pl.ds(base, CH)], rows)
            pltpu.sync_copy(rows, acc.at[idxv], add=True)

        plsc.subcore_barrier()
        pltpu.sync_copy(acc.at[pl.ds(s * NPS, NPS)],
                        out_h.at[c].at[pl.ds(s * NPS, NPS)])

    return k(edge_out, row)


def kernel(x, h, edge_index, edge_fea, W_emb, b_emb, params):
    row = edge_index[0].astype(jnp.int32)
    col = edge_index[1].astype(jnp.int32)
    h = _embed(h, W_emb, b_emb)
    equ = jnp.pad(x, ((0, 0), (0, EQW - 3)))
    for p in params:
        A, B, gate = _node_pre(h, p)
        A144 = jnp.concatenate([A, equ], axis=1)
        B144 = jnp.concatenate([B, equ], axis=1)
        pre1, rij = _sc_gather(A144, B144, row, col)
        edge_out = _edge_mlp(pre1, rij, edge_fea, p)
        agg = _sc_scatter(edge_out, row)
        h, equ = _node_update(h, equ, gate, agg, p)
    return equ[:, :3]


# speedup vs baseline: 1.0984x; 1.0830x over previous
import functools
import jax
import jax.numpy as jnp
from jax import lax
from jax.experimental import pallas as pl
from jax.experimental.pallas import tpu as pltpu
from jax.experimental.pallas import tpu_sc as plsc

N_NODES = 10000
N_EDGES = 320000
H = 128
DE = 16
EQW = 16
TAILW = 16
OUTW = H + TAILW

BN = 1000
BE = 1280


def _silu(x):
    return x * jax.nn.sigmoid(x)


def _node_pre_body(h_ref, Wr_ref, Wc_ref, Wg1_ref, bg1_ref, Wg2_ref, bg2_ref,
                   A_ref, B_ref, gate_ref):
    h = h_ref[...]
    A_ref[...] = jnp.dot(h, Wr_ref[...], preferred_element_type=jnp.float32)
    B_ref[...] = jnp.dot(h, Wc_ref[...], preferred_element_type=jnp.float32)
    g1 = _silu(jnp.dot(h, Wg1_ref[...], preferred_element_type=jnp.float32)
               + bg1_ref[...])
    gate_ref[...] = (jnp.dot(g1, Wg2_ref[...], preferred_element_type=jnp.float32)
                     + bg2_ref[...])


def _node_pre(h, p):
    Wr = p['edge']['W1'][1:1 + H]
    Wc = p['edge']['W1'][1 + H:1 + 2 * H]
    Wg1 = p['node_equ']['W1']
    bg1 = p['node_equ']['b1'][None, :]
    Wg2 = p['node_equ']['W2']
    bg2 = p['node_equ']['b2'][None, :]
    grid = (N_NODES // BN,)
    full = lambda r, c: pl.BlockSpec((r, c), lambda i: (0, 0))
    blk = lambda c: pl.BlockSpec((BN, c), lambda i: (i, 0))
    return pl.pallas_call(
        _node_pre_body,
        grid=grid,
        in_specs=[blk(H), full(H, H), full(H, H), full(H, H), full(1, H),
                  full(H, 1), full(1, 1)],
        out_specs=[blk(H), blk(H), blk(1)],
        out_shape=[jax.ShapeDtypeStruct((N_NODES, H), jnp.float32),
                   jax.ShapeDtypeStruct((N_NODES, H), jnp.float32),
                   jax.ShapeDtypeStruct((N_NODES, 1), jnp.float32)],
    )(h, Wr, Wc, Wg1, bg1, Wg2, bg2)


def _embed_body(h_ref, We_ref, be_ref, out_ref):
    out_ref[...] = (jnp.dot(h_ref[...], We_ref[...],
                            preferred_element_type=jnp.float32) + be_ref[...])


def _embed(h, W_emb, b_emb):
    grid = (N_NODES // BN,)
    return pl.pallas_call(
        _embed_body,
        grid=grid,
        in_specs=[pl.BlockSpec((BN, H), lambda i: (i, 0)),
                  pl.BlockSpec((H, H), lambda i: (0, 0)),
                  pl.BlockSpec((1, H), lambda i: (0, 0))],
        out_specs=pl.BlockSpec((BN, H), lambda i: (i, 0)),
        out_shape=jax.ShapeDtypeStruct((N_NODES, H), jnp.float32),
    )(h, W_emb, b_emb[None, :])


def _edge_body(pre1_ref, rij_ref, ef_ref, w1s_ref, W1e_ref, b1_ref,
               W2_ref, b2_ref, Wc1_ref, bc1_ref, Wc2_ref, bc2_ref, out_ref):
    blk = pl.program_id(0)

    @pl.when(blk >= N_EDGES // BE)
    def _():
        out_ref[...] = jnp.zeros_like(out_ref)

    @pl.when(blk < N_EDGES // BE)
    def _():
        _edge_compute(pre1_ref, rij_ref, ef_ref, w1s_ref, W1e_ref, b1_ref,
                      W2_ref, b2_ref, Wc1_ref, bc1_ref, Wc2_ref, bc2_ref,
                      out_ref)


def _edge_compute(pre1_ref, rij_ref, ef_ref, w1s_ref, W1e_ref, b1_ref,
                  W2_ref, b2_ref, Wc1_ref, bc1_ref, Wc2_ref, bc2_ref, out_ref):
    rij = rij_ref[...]
    s2 = jnp.sum(rij * rij, axis=1, keepdims=True) + 1e-12
    sij = jnp.sqrt(s2)
    z = (pre1_ref[...]
         + jnp.dot(ef_ref[...], W1e_ref[...], preferred_element_type=jnp.float32)
         + sij * w1s_ref[...] + b1_ref[...])
    u = _silu(z)
    m = _silu(jnp.dot(u, W2_ref[...], preferred_element_type=jnp.float32)
              + b2_ref[...])
    v = _silu(jnp.dot(m, Wc1_ref[...], preferred_element_type=jnp.float32)
              + bc1_ref[...])
    cm = (jnp.dot(v, Wc2_ref[...], preferred_element_type=jnp.float32)
          + bc2_ref[...])
    ones_col = (jax.lax.broadcasted_iota(jnp.int32, (1, TAILW), 1) == 3
                ).astype(jnp.float32)
    tail = rij * cm + ones_col
    out_ref[...] = jnp.concatenate([m, tail], axis=1)


def _edge_mlp(pre1, rij, edge_fea, p):
    w1s = p['edge']['W1'][0:1]
    W1e = p['edge']['W1'][1 + 2 * H:]
    b1 = p['edge']['b1'][None, :]
    W2 = p['edge']['W2']
    b2 = p['edge']['b2'][None, :]
    Wc1 = p['coord']['W1']
    bc1 = p['coord']['b1'][None, :]
    Wc2 = p['coord']['W2']
    bc2 = p['coord']['b2'][None, :]
    grid = (E_PAD // BE,)
    full = lambda r, c: pl.BlockSpec((r, c), lambda i: (0, 0))
    blk = lambda c: pl.BlockSpec((BE, c), lambda i: (i, 0))
    return pl.pallas_call(
        _edge_body,
        grid=grid,
        in_specs=[blk(H), blk(EQW), blk(DE), full(1, H), full(DE, H),
                  full(1, H), full(H, H), full(1, H), full(H, H), full(1, H),
                  full(H, 1), full(1, 1)],
        out_specs=blk(OUTW),
        out_shape=jax.ShapeDtypeStruct((E_PAD, OUTW), jnp.float32),
    )(pre1, rij, edge_fea, w1s, W1e, b1, W2, b2, Wc1, bc1, Wc2, bc2)


def _node_upd_body(h_ref, equ_ref, gate_ref, agg_ref,
                   wequ_ref, Wn1h_ref, Wn1m_ref, bn1_ref, Wn2_ref, bn2_ref,
                   h_out_ref, equ_out_ref):
    agg = jnp.sum(agg_ref[...], axis=0)
    msum = agg[:, :H]
    tail = agg[:, H:]
    cnt = tail[:, 3:4]
    fmean = tail / jnp.maximum(cnt, 1.0)
    mask = (jax.lax.broadcasted_iota(jnp.int32, (1, EQW), 1) < 3
            ).astype(jnp.float32)
    equ_out_ref[...] = (equ_ref[...] * wequ_ref[0, 0] * gate_ref[...]
                        + fmean) * mask
    h = h_ref[...]
    u = _silu(jnp.dot(h, Wn1h_ref[...], preferred_element_type=jnp.float32)
              + jnp.dot(msum, Wn1m_ref[...], preferred_element_type=jnp.float32)
              + bn1_ref[...])
    h_out_ref[...] = (jnp.dot(u, Wn2_ref[...], preferred_element_type=jnp.float32)
                      + bn2_ref[...] + h)


def _node_update(h, equ, gate, agg_slabs, p):
    Wn1h = p['node']['W1'][:H]
    Wn1m = p['node']['W1'][H:]
    bn1 = p['node']['b1'][None, :]
    Wn2 = p['node']['W2']
    bn2 = p['node']['b2'][None, :]
    S = agg_slabs.shape[0]
    grid = (N_NODES // BN,)
    full = lambda r, c: pl.BlockSpec((r, c), lambda i: (0, 0))
    blk = lambda c: pl.BlockSpec((BN, c), lambda i: (i, 0))
    return pl.pallas_call(
        _node_upd_body,
        grid=grid,
        in_specs=[blk(H), blk(EQW), blk(1),
                  pl.BlockSpec((S, BN, OUTW), lambda i: (0, i, 0)),
                  full(1, 1), full(H, H), full(H, H), full(1, H), full(H, H),
                  full(1, H)],
        out_specs=[blk(H), blk(EQW)],
        out_shape=[jax.ShapeDtypeStruct((N_NODES, H), jnp.float32),
                   jax.ShapeDtypeStruct((N_NODES, EQW), jnp.float32)],
    )(h, equ, gate, agg_slabs, p['W_equ'], Wn1h, Wn1m, bn1, Wn2, bn2)


NW = 32
CH = 128
CHG = 64
E_PAD = NW * 80 * CH
TS = E_PAD // (NW * CH)
TG = E_PAD // (NW * CHG)
EPW = E_PAD // NW
NPS = N_NODES // 16


AW = H + EQW


def _sc_gather(A144, B144, row, col):
    mesh = plsc.VectorSubcoreMesh(core_axis_name="c", subcore_axis_name="s")

    @functools.partial(
        pl.kernel, mesh=mesh,
        compiler_params=pltpu.CompilerParams(use_tc_tiling_on_sc=False),
        out_type=[jax.ShapeDtypeStruct((E_PAD, H), jnp.float32),
                  jax.ShapeDtypeStruct((E_PAD, EQW), jnp.float32)],
        scratch_types=[pltpu.VMEM((EPW,), jnp.int32),
                       pltpu.VMEM((EPW,), jnp.int32),
                       pltpu.VMEM((CHG, AW), jnp.float32),
                       pltpu.VMEM((CHG, AW), jnp.float32),
                       pltpu.VMEM((CHG, AW), jnp.float32),
                       pltpu.VMEM((CHG, AW), jnp.float32),
                       pltpu.VMEM((CHG, H), jnp.float32),
                       pltpu.VMEM((CHG, H), jnp.float32),
                       pltpu.VMEM((CHG, EQW), jnp.float32),
                       pltpu.VMEM((CHG, EQW), jnp.float32),
                       pltpu.SemaphoreType.DMA,
                       pltpu.SemaphoreType.DMA,
                       pltpu.SemaphoreType.DMA,
                       pltpu.SemaphoreType.DMA],
    )
    def k(A_h, B_h, row_h, col_h, pre1_h, rij_h,
          rva, cva, av0, av1, bv0, bv1, p0, p1, r0, r1,
          sg0, sg1, sw0, sw1):
        c = lax.axis_index("c")
        s = lax.axis_index("s")
        wid = s * 2 + c
        ebase = wid * EPW
        pltpu.sync_copy(row_h.at[pl.ds(ebase, EPW)], rva)
        pltpu.sync_copy(col_h.at[pl.ds(ebase, EPW)], cva)

        sets = ((av0, bv0, p0, r0, sg0, sw0),
                (av1, bv1, p1, r1, sg1, sw1))

        def g_copies(st, t):
            av, bv, _, _, sg, _ = st
            rs = rva.at[pl.ds(t * CHG, CHG)]
            cs = cva.at[pl.ds(t * CHG, CHG)]
            return (pltpu.make_async_copy(A_h.at[rs], av, sg),
                    pltpu.make_async_copy(B_h.at[cs], bv, sg))

        def w_copies(st, t):
            _, _, p, r, _, sw = st
            dst = ebase + t * CHG
            return (pltpu.make_async_copy(p, pre1_h.at[pl.ds(dst, CHG)], sw),
                    pltpu.make_async_copy(r, rij_h.at[pl.ds(dst, CHG)], sw))

        def slot(si, t):
            st = sets[si]
            av, bv, p, r, _, _ = st
            for cp in g_copies(st, t):
                cp.wait()
            @pl.when(t >= 2)
            def _():
                for cp in w_copies(st, t - 2):
                    cp.wait()

            @pl.loop(0, CHG)
            def _(i):
                for j in range(0, H, 16):
                    p.at[i, pl.ds(j, 16)][...] = (
                        av.at[i, pl.ds(j, 16)][...]
                        + bv.at[i, pl.ds(j, 16)][...])
                r.at[i][...] = (av.at[i, pl.ds(H, EQW)][...]
                                - bv.at[i, pl.ds(H, EQW)][...])

            for cp in w_copies(st, t):
                cp.start()
            @pl.when(t + 2 < TG)
            def _():
                for cp in g_copies(st, t + 2):
                    cp.start()

        for cp in g_copies(sets[0], 0):
            cp.start()
        for cp in g_copies(sets[1], 1):
            cp.start()

        @pl.loop(0, TG, step=2)
        def _(t):
            slot(0, t)
            slot(1, t + 1)

        for cp in w_copies(sets[0], TG - 2):
            cp.wait()
        for cp in w_copies(sets[1], TG - 1):
            cp.wait()

    return k(A144, B144, row, col)


def _sc_scatter(edge_out, row):
    mesh = plsc.VectorSubcoreMesh(core_axis_name="c", subcore_axis_name="s")

    @functools.partial(
        pl.kernel, mesh=mesh,
        compiler_params=pltpu.CompilerParams(use_tc_tiling_on_sc=False),
        out_type=jax.ShapeDtypeStruct((2, N_NODES, OUTW), jnp.float32),
        scratch_types=[pltpu.VMEM((CH,), jnp.int32),
                       pltpu.VMEM((CH, OUTW), jnp.float32),
                       pltpu.VMEM_SHARED((N_NODES, OUTW), jnp.float32)],
    )
    def k(eo_h, row_h, out_h, idxv, rows, acc):
        c = lax.axis_index("c")
        s = lax.axis_index("s")
        wid = s * 2 + c
        cbase = wid * TS

        @pl.loop(0, 125)
        def _(i):
            @pl.loop(0, OUTW, step=16)
            def _(j):
                rows.at[i, pl.ds(j, 16)][...] = jnp.zeros((16,), jnp.float32)

        @pl.loop(0, NPS, step=125)
        def _(r):
            pltpu.sync_copy(rows.at[pl.ds(0, 125)],
                            acc.at[pl.ds(s * NPS + r, 125)])

        plsc.subcore_barrier()

        @pl.loop(cbase, cbase + TS)
        def _(chk):
            base = chk * CH
            pltpu.sync_copy(row_h.at[pl.ds(base, CH)], idxv)
            pltpu.sync_copy(eo_h.at[pl.ds(base, CH)], rows)
            pltpu.sync_copy(rows, acc.at[idxv], add=True)

        plsc.subcore_barrier()
        pltpu.sync_copy(acc.at[pl.ds(s * NPS, NPS)],
                        out_h.at[c].at[pl.ds(s * NPS, NPS)])

    return k(edge_out, row)


def kernel(x, h, edge_index, edge_fea, W_emb, b_emb, params):
    pad_e = E_PAD - N_EDGES
    row = jnp.pad(edge_index[0].astype(jnp.int32), (0, pad_e))
    col = jnp.pad(edge_index[1].astype(jnp.int32), (0, pad_e))
    edge_fea = jnp.pad(edge_fea, ((0, pad_e), (0, 0)))
    h = _embed(h, W_emb, b_emb)
    equ = jnp.pad(x, ((0, 0), (0, EQW - 3)))
    for p in params:
        A, B, gate = _node_pre(h, p)
        A144 = jnp.concatenate([A, equ], axis=1)
        B144 = jnp.concatenate([B, equ], axis=1)
        pre1, rij = _sc_gather(A144, B144, row, col)
        edge_out = _edge_mlp(pre1, rij, edge_fea, p)
        agg = _sc_scatter(edge_out, row)
        h, equ = _node_update(h, equ, gate, agg, p)
    return equ[:, :3]


# speedup vs baseline: 1.2380x; 1.1271x over previous
import functools
import jax
import jax.numpy as jnp
from jax import lax
from jax.experimental import pallas as pl
from jax.experimental.pallas import tpu as pltpu
from jax.experimental.pallas import tpu_sc as plsc

N_NODES = 10000
N_EDGES = 320000
H = 128
DE = 16
EQW = 16
TAILW = 16
OUTW = H + TAILW

BN = 1000
BE = 1280


def _silu(x):
    return x * jax.nn.sigmoid(x)


def _node_pre_body(h_ref, Wr_ref, Wc_ref, Wg1_ref, bg1_ref, Wg2_ref, bg2_ref,
                   A_ref, B_ref, gate_ref):
    h = h_ref[...]
    A_ref[...] = jnp.dot(h, Wr_ref[...], preferred_element_type=jnp.float32)
    B_ref[...] = jnp.dot(h, Wc_ref[...], preferred_element_type=jnp.float32)
    g1 = _silu(jnp.dot(h, Wg1_ref[...], preferred_element_type=jnp.float32)
               + bg1_ref[...])
    gate_ref[...] = (jnp.dot(g1, Wg2_ref[...], preferred_element_type=jnp.float32)
                     + bg2_ref[...])


def _node_pre(h, p):
    Wr = p['edge']['W1'][1:1 + H]
    Wc = p['edge']['W1'][1 + H:1 + 2 * H]
    Wg1 = p['node_equ']['W1']
    bg1 = p['node_equ']['b1'][None, :]
    Wg2 = p['node_equ']['W2']
    bg2 = p['node_equ']['b2'][None, :]
    grid = (N_NODES // BN,)
    full = lambda r, c: pl.BlockSpec((r, c), lambda i: (0, 0))
    blk = lambda c: pl.BlockSpec((BN, c), lambda i: (i, 0))
    return pl.pallas_call(
        _node_pre_body,
        grid=grid,
        in_specs=[blk(H), full(H, H), full(H, H), full(H, H), full(1, H),
                  full(H, 1), full(1, 1)],
        out_specs=[blk(H), blk(H), blk(1)],
        out_shape=[jax.ShapeDtypeStruct((N_NODES, H), jnp.float32),
                   jax.ShapeDtypeStruct((N_NODES, H), jnp.float32),
                   jax.ShapeDtypeStruct((N_NODES, 1), jnp.float32)],
    )(h, Wr, Wc, Wg1, bg1, Wg2, bg2)


def _embed_body(h_ref, We_ref, be_ref, out_ref):
    out_ref[...] = (jnp.dot(h_ref[...], We_ref[...],
                            preferred_element_type=jnp.float32) + be_ref[...])


def _embed(h, W_emb, b_emb):
    grid = (N_NODES // BN,)
    return pl.pallas_call(
        _embed_body,
        grid=grid,
        in_specs=[pl.BlockSpec((BN, H), lambda i: (i, 0)),
                  pl.BlockSpec((H, H), lambda i: (0, 0)),
                  pl.BlockSpec((1, H), lambda i: (0, 0))],
        out_specs=pl.BlockSpec((BN, H), lambda i: (i, 0)),
        out_shape=jax.ShapeDtypeStruct((N_NODES, H), jnp.float32),
    )(h, W_emb, b_emb[None, :])


def _edge_body(pre1_ref, rij_ref, ef_ref, w1s_ref, W1e_ref, b1_ref,
               W2_ref, b2_ref, Wc1_ref, bc1_ref, Wc2_ref, bc2_ref,
               m_ref, tail_ref):
    blk = pl.program_id(0)

    @pl.when(blk >= N_EDGES // BE)
    def _():
        m_ref[...] = jnp.zeros_like(m_ref)
        tail_ref[...] = jnp.zeros_like(tail_ref)

    @pl.when(blk < N_EDGES // BE)
    def _():
        _edge_compute(pre1_ref, rij_ref, ef_ref, w1s_ref, W1e_ref, b1_ref,
                      W2_ref, b2_ref, Wc1_ref, bc1_ref, Wc2_ref, bc2_ref,
                      m_ref, tail_ref)


def _edge_compute(pre1_ref, rij_ref, ef_ref, w1s_ref, W1e_ref, b1_ref,
                  W2_ref, b2_ref, Wc1_ref, bc1_ref, Wc2_ref, bc2_ref,
                  m_ref, tail_ref):
    rij = rij_ref[...]
    s2 = jnp.sum(rij * rij, axis=1, keepdims=True) + 1e-12
    sij = jnp.sqrt(s2)
    z = (pre1_ref[...]
         + jnp.dot(ef_ref[...], W1e_ref[...], preferred_element_type=jnp.float32)
         + sij * w1s_ref[...] + b1_ref[...])
    u = _silu(z).astype(jnp.bfloat16)
    m = _silu(jnp.dot(u, W2_ref[...], preferred_element_type=jnp.float32)
              + b2_ref[...])
    v = _silu(jnp.dot(m.astype(jnp.bfloat16), Wc1_ref[...],
                      preferred_element_type=jnp.float32)
              + bc1_ref[...]).astype(jnp.bfloat16)
    cm = (jnp.dot(v, Wc2_ref[...], preferred_element_type=jnp.float32)
          + bc2_ref[...])
    ones_col = (jax.lax.broadcasted_iota(jnp.int32, (1, TAILW), 1) == 3
                ).astype(jnp.float32)
    m_ref[...] = m
    tail_ref[...] = rij * cm + ones_col


def _edge_mlp(pre1, rij, edge_fea, p):
    w1s = p['edge']['W1'][0:1]
    W1e = p['edge']['W1'][1 + 2 * H:]
    b1 = p['edge']['b1'][None, :]
    W2 = p['edge']['W2'].astype(jnp.bfloat16)
    b2 = p['edge']['b2'][None, :]
    Wc1 = p['coord']['W1'].astype(jnp.bfloat16)
    bc1 = p['coord']['b1'][None, :]
    Wc2 = p['coord']['W2'].astype(jnp.bfloat16)
    bc2 = p['coord']['b2'][None, :]
    grid = (E_PAD // BE,)
    nreal = N_EDGES // BE - 1
    full = lambda r, c: pl.BlockSpec((r, c), lambda i: (0, 0))
    blk = lambda c: pl.BlockSpec((BE, c), lambda i: (i, 0))
    efspec = pl.BlockSpec((BE, DE), lambda i: (jnp.minimum(i, nreal), 0))
    return pl.pallas_call(
        _edge_body,
        grid=grid,
        in_specs=[blk(H), blk(EQW), efspec, full(1, H), full(DE, H),
                  full(1, H), full(H, H), full(1, H), full(H, H), full(1, H),
                  full(H, 1), full(1, 1)],
        out_specs=[blk(H), blk(TAILW)],
        out_shape=[jax.ShapeDtypeStruct((E_PAD, H), jnp.float32),
                   jax.ShapeDtypeStruct((E_PAD, TAILW), jnp.float32)],
    )(pre1, rij, edge_fea, w1s, W1e, b1, W2, b2, Wc1, bc1, Wc2, bc2)


def _node_upd_body(h_ref, equ_ref, gate_ref, aggm_ref, aggt_ref,
                   wequ_ref, Wn1h_ref, Wn1m_ref, bn1_ref, Wn2_ref, bn2_ref,
                   h_out_ref, equ_out_ref):
    msum = jnp.sum(aggm_ref[...], axis=0)
    tail = jnp.sum(aggt_ref[...], axis=0)
    cnt = tail[:, 3:4]
    fmean = tail / jnp.maximum(cnt, 1.0)
    mask = (jax.lax.broadcasted_iota(jnp.int32, (1, EQW), 1) < 3
            ).astype(jnp.float32)
    equ_out_ref[...] = (equ_ref[...] * wequ_ref[0, 0] * gate_ref[...]
                        + fmean) * mask
    h = h_ref[...]
    u = _silu(jnp.dot(h, Wn1h_ref[...], preferred_element_type=jnp.float32)
              + jnp.dot(msum, Wn1m_ref[...], preferred_element_type=jnp.float32)
              + bn1_ref[...])
    h_out_ref[...] = (jnp.dot(u, Wn2_ref[...], preferred_element_type=jnp.float32)
                      + bn2_ref[...] + h)


def _node_update(h, equ, gate, aggm, aggt, p):
    Wn1h = p['node']['W1'][:H]
    Wn1m = p['node']['W1'][H:]
    bn1 = p['node']['b1'][None, :]
    Wn2 = p['node']['W2']
    bn2 = p['node']['b2'][None, :]
    S = aggm.shape[0]
    grid = (N_NODES // BN,)
    full = lambda r, c: pl.BlockSpec((r, c), lambda i: (0, 0))
    blk = lambda c: pl.BlockSpec((BN, c), lambda i: (i, 0))
    return pl.pallas_call(
        _node_upd_body,
        grid=grid,
        in_specs=[blk(H), blk(EQW), blk(1),
                  pl.BlockSpec((S, BN, H), lambda i: (0, i, 0)),
                  pl.BlockSpec((S, BN, TAILW), lambda i: (0, i, 0)),
                  full(1, 1), full(H, H), full(H, H), full(1, H), full(H, H),
                  full(1, H)],
        out_specs=[blk(H), blk(EQW)],
        out_shape=[jax.ShapeDtypeStruct((N_NODES, H), jnp.float32),
                   jax.ShapeDtypeStruct((N_NODES, EQW), jnp.float32)],
    )(h, equ, gate, aggm, aggt, p['W_equ'], Wn1h, Wn1m, bn1, Wn2, bn2)


NW = 32
CH = 128
CHG = 64
E_PAD = NW * 80 * CH
TS = E_PAD // (NW * CH)
TG = E_PAD // (NW * CHG)
EPW = E_PAD // NW
NPS = N_NODES // 16


AW = H + EQW


def _sc_gather(A144, B144, row, col):
    mesh = plsc.VectorSubcoreMesh(core_axis_name="c", subcore_axis_name="s")

    @functools.partial(
        pl.kernel, mesh=mesh,
        compiler_params=pltpu.CompilerParams(use_tc_tiling_on_sc=False),
        out_type=[jax.ShapeDtypeStruct((E_PAD, H), jnp.float32),
                  jax.ShapeDtypeStruct((E_PAD, EQW), jnp.float32)],
        scratch_types=[pltpu.VMEM((EPW,), jnp.int32),
                       pltpu.VMEM((EPW,), jnp.int32),
                       pltpu.VMEM((CHG, AW), jnp.float32),
                       pltpu.VMEM((CHG, AW), jnp.float32),
                       pltpu.VMEM((CHG, AW), jnp.float32),
                       pltpu.VMEM((CHG, AW), jnp.float32),
                       pltpu.VMEM((CHG, H), jnp.float32),
                       pltpu.VMEM((CHG, H), jnp.float32),
                       pltpu.VMEM((CHG, EQW), jnp.float32),
                       pltpu.VMEM((CHG, EQW), jnp.float32),
                       pltpu.SemaphoreType.DMA,
                       pltpu.SemaphoreType.DMA,
                       pltpu.SemaphoreType.DMA,
                       pltpu.SemaphoreType.DMA],
    )
    def k(A_h, B_h, row_h, col_h, pre1_h, rij_h,
          rva, cva, av0, av1, bv0, bv1, p0, p1, r0, r1,
          sg0, sg1, sw0, sw1):
        c = lax.axis_index("c")
        s = lax.axis_index("s")
        wid = s * 2 + c
        ebase = wid * EPW
        pltpu.sync_copy(row_h.at[pl.ds(ebase, EPW)], rva)
        pltpu.sync_copy(col_h.at[pl.ds(ebase, EPW)], cva)

        sets = ((av0, bv0, p0, r0, sg0, sw0),
                (av1, bv1, p1, r1, sg1, sw1))

        def g_copies(st, t):
            av, bv, _, _, sg, _ = st
            rs = rva.at[pl.ds(t * CHG, CHG)]
            cs = cva.at[pl.ds(t * CHG, CHG)]
            return (pltpu.make_async_copy(A_h.at[rs], av, sg),
                    pltpu.make_async_copy(B_h.at[cs], bv, sg))

        def w_copies(st, t):
            _, _, p, r, _, sw = st
            dst = ebase + t * CHG
            return (pltpu.make_async_copy(p, pre1_h.at[pl.ds(dst, CHG)], sw),
                    pltpu.make_async_copy(r, rij_h.at[pl.ds(dst, CHG)], sw))

        def slot(si, t):
            st = sets[si]
            av, bv, p, r, _, _ = st
            for cp in g_copies(st, t):
                cp.wait()
            @pl.when(t >= 2)
            def _():
                for cp in w_copies(st, t - 2):
                    cp.wait()

            @pl.loop(0, CHG)
            def _(i):
                for j in range(0, H, 16):
                    p.at[i, pl.ds(j, 16)][...] = (
                        av.at[i, pl.ds(j, 16)][...]
                        + bv.at[i, pl.ds(j, 16)][...])
                r.at[i][...] = (av.at[i, pl.ds(H, EQW)][...]
                                - bv.at[i, pl.ds(H, EQW)][...])

            for cp in w_copies(st, t):
                cp.start()
            @pl.when(t + 2 < TG)
            def _():
                for cp in g_copies(st, t + 2):
                    cp.start()

        for cp in g_copies(sets[0], 0):
            cp.start()
        for cp in g_copies(sets[1], 1):
            cp.start()

        @pl.loop(0, TG, step=2)
        def _(t):
            slot(0, t)
            slot(1, t + 1)

        for cp in w_copies(sets[0], TG - 2):
            cp.wait()
        for cp in w_copies(sets[1], TG - 1):
            cp.wait()

    return k(A144, B144, row, col)


def _sc_scatter(m_out, tail_out, row):
    mesh = plsc.VectorSubcoreMesh(core_axis_name="c", subcore_axis_name="s")

    @functools.partial(
        pl.kernel, mesh=mesh,
        compiler_params=pltpu.CompilerParams(use_tc_tiling_on_sc=False),
        out_type=[jax.ShapeDtypeStruct((2, N_NODES, H), jnp.float32),
                  jax.ShapeDtypeStruct((2, N_NODES, TAILW), jnp.float32)],
        scratch_types=[pltpu.VMEM((CH,), jnp.int32),
                       pltpu.VMEM((CH, H), jnp.float32),
                       pltpu.VMEM((CH, TAILW), jnp.float32),
                       pltpu.VMEM_SHARED((N_NODES, H), jnp.float32),
                       pltpu.VMEM_SHARED((N_NODES, TAILW), jnp.float32)],
    )
    def k(m_h, t_h, row_h, outm_h, outt_h, idxv, mrows, trows, accm, acct):
        c = lax.axis_index("c")
        s = lax.axis_index("s")
        wid = s * 2 + c
        cbase = wid * TS

        @pl.loop(0, 125)
        def _(i):
            for j in range(0, H, 16):
                mrows.at[i, pl.ds(j, 16)][...] = jnp.zeros((16,), jnp.float32)
            trows.at[i][...] = jnp.zeros((16,), jnp.float32)

        @pl.loop(0, NPS, step=125)
        def _(r):
            pltpu.sync_copy(mrows.at[pl.ds(0, 125)],
                            accm.at[pl.ds(s * NPS + r, 125)])
            pltpu.sync_copy(trows.at[pl.ds(0, 125)],
                            acct.at[pl.ds(s * NPS + r, 125)])

        plsc.subcore_barrier()

        @pl.loop(cbase, cbase + TS)
        def _(chk):
            base = chk * CH
            pltpu.sync_copy(row_h.at[pl.ds(base, CH)], idxv)
            pltpu.sync_copy(m_h.at[pl.ds(base, CH)], mrows)
            pltpu.sync_copy(t_h.at[pl.ds(base, CH)], trows)
            pltpu.sync_copy(mrows, accm.at[idxv], add=True)
            pltpu.sync_copy(trows, acct.at[idxv], add=True)

        plsc.subcore_barrier()
        pltpu.sync_copy(accm.at[pl.ds(s * NPS, NPS)],
                        outm_h.at[c].at[pl.ds(s * NPS, NPS)])
        pltpu.sync_copy(acct.at[pl.ds(s * NPS, NPS)],
                        outt_h.at[c].at[pl.ds(s * NPS, NPS)])

    return k(m_out, tail_out, row)


def kernel(x, h, edge_index, edge_fea, W_emb, b_emb, params):
    pad_e = E_PAD - N_EDGES
    row = jnp.pad(edge_index[0].astype(jnp.int32), (0, pad_e))
    col = jnp.pad(edge_index[1].astype(jnp.int32), (0, pad_e))
    h = _embed(h, W_emb, b_emb)
    equ = jnp.pad(x, ((0, 0), (0, EQW - 3)))
    for p in params:
        A, B, gate = _node_pre(h, p)
        A144 = jnp.concatenate([A, equ], axis=1)
        B144 = jnp.concatenate([B, equ], axis=1)
        pre1, rij = _sc_gather(A144, B144, row, col)
        m_out, tail_out = _edge_mlp(pre1, rij, edge_fea, p)
        aggm, aggt = _sc_scatter(m_out, tail_out, row)
        h, equ = _node_update(h, equ, gate, aggm, aggt, p)
    return equ[:, :3]


# speedup vs baseline: 1.3689x; 1.1057x over previous
import functools
import jax
import jax.numpy as jnp
from jax import lax
from jax.experimental import pallas as pl
from jax.experimental.pallas import tpu as pltpu
from jax.experimental.pallas import tpu_sc as plsc

N_NODES = 10000
N_EDGES = 320000
H = 128
DE = 16
EQW = 16
TAILW = 16
OUTW = H + TAILW

BN = 1000
BE = 1280


def _silu(x):
    return x / (1.0 + jnp.exp(-x))


def _node_pre_body(h_ref, Wr_ref, Wc_ref, Wg1_ref, bg1_ref, Wg2_ref, bg2_ref,
                   A_ref, B_ref, gate_ref):
    h = h_ref[...]
    A_ref[...] = jnp.dot(h, Wr_ref[...], preferred_element_type=jnp.float32)
    B_ref[...] = jnp.dot(h, Wc_ref[...], preferred_element_type=jnp.float32)
    g1 = _silu(jnp.dot(h, Wg1_ref[...], preferred_element_type=jnp.float32)
               + bg1_ref[...])
    gate_ref[...] = (jnp.dot(g1, Wg2_ref[...], preferred_element_type=jnp.float32)
                     + bg2_ref[...])


def _node_pre(h, p):
    Wr = p['edge']['W1'][1:1 + H]
    Wc = p['edge']['W1'][1 + H:1 + 2 * H]
    Wg1 = p['node_equ']['W1']
    bg1 = p['node_equ']['b1'][None, :]
    Wg2 = p['node_equ']['W2']
    bg2 = p['node_equ']['b2'][None, :]
    grid = (N_NODES // BN,)
    full = lambda r, c: pl.BlockSpec((r, c), lambda i: (0, 0))
    blk = lambda c: pl.BlockSpec((BN, c), lambda i: (i, 0))
    return pl.pallas_call(
        _node_pre_body,
        grid=grid,
        in_specs=[blk(H), full(H, H), full(H, H), full(H, H), full(1, H),
                  full(H, 1), full(1, 1)],
        out_specs=[blk(H), blk(H), blk(1)],
        out_shape=[jax.ShapeDtypeStruct((N_NODES, H), jnp.float32),
                   jax.ShapeDtypeStruct((N_NODES, H), jnp.float32),
                   jax.ShapeDtypeStruct((N_NODES, 1), jnp.float32)],
    )(h, Wr, Wc, Wg1, bg1, Wg2, bg2)


def _embed_body(h_ref, We_ref, be_ref, out_ref):
    out_ref[...] = (jnp.dot(h_ref[...], We_ref[...],
                            preferred_element_type=jnp.float32) + be_ref[...])


def _embed(h, W_emb, b_emb):
    grid = (N_NODES // BN,)
    return pl.pallas_call(
        _embed_body,
        grid=grid,
        in_specs=[pl.BlockSpec((BN, H), lambda i: (i, 0)),
                  pl.BlockSpec((H, H), lambda i: (0, 0)),
                  pl.BlockSpec((1, H), lambda i: (0, 0))],
        out_specs=pl.BlockSpec((BN, H), lambda i: (i, 0)),
        out_shape=jax.ShapeDtypeStruct((N_NODES, H), jnp.float32),
    )(h, W_emb, b_emb[None, :])


def _edge_body(pre1_ref, rij_ref, ef_ref, w1s_ref, W1e_ref, b1_ref,
               W2_ref, b2_ref, Wc1_ref, bc1_ref, Wc2_ref, bc2_ref,
               m_ref, tail_ref):
    blk = pl.program_id(0)

    @pl.when(blk >= N_EDGES // BE)
    def _():
        m_ref[...] = jnp.zeros_like(m_ref)
        tail_ref[...] = jnp.zeros_like(tail_ref)

    @pl.when(blk < N_EDGES // BE)
    def _():
        _edge_compute(pre1_ref, rij_ref, ef_ref, w1s_ref, W1e_ref, b1_ref,
                      W2_ref, b2_ref, Wc1_ref, bc1_ref, Wc2_ref, bc2_ref,
                      m_ref, tail_ref)


def _edge_compute(pre1_ref, rij_ref, ef_ref, w1s_ref, W1e_ref, b1_ref,
                  W2_ref, b2_ref, Wc1_ref, bc1_ref, Wc2_ref, bc2_ref,
                  m_ref, tail_ref):
    rij = rij_ref[:, 0:EQW]
    s2 = jnp.sum(rij * rij, axis=1, keepdims=True) + 1e-12
    sij = jnp.sqrt(s2)
    z = (pre1_ref[...]
         + jnp.dot(ef_ref[...], W1e_ref[...], preferred_element_type=jnp.float32)
         + sij * w1s_ref[...] + b1_ref[...])
    u = _silu(z).astype(jnp.bfloat16)
    m = _silu(jnp.dot(u, W2_ref[...], preferred_element_type=jnp.float32)
              + b2_ref[...])
    v = _silu(jnp.dot(m.astype(jnp.bfloat16), Wc1_ref[...],
                      preferred_element_type=jnp.float32)
              + bc1_ref[...]).astype(jnp.bfloat16)
    cm = (jnp.dot(v, Wc2_ref[...], preferred_element_type=jnp.float32)
          + bc2_ref[...])
    ones_col = (jax.lax.broadcasted_iota(jnp.int32, (1, TAILW), 1) == 3
                ).astype(jnp.float32)
    m_ref[...] = m
    tail = rij * cm + ones_col
    tail_ref[...] = jnp.concatenate(
        [tail, jnp.zeros((tail.shape[0], H - TAILW), jnp.float32)], axis=1)


def _edge_mlp(pre1, rij, edge_fea, p):
    w1s = p['edge']['W1'][0:1]
    W1e = p['edge']['W1'][1 + 2 * H:]
    b1 = p['edge']['b1'][None, :]
    W2 = p['edge']['W2'].astype(jnp.bfloat16)
    b2 = p['edge']['b2'][None, :]
    Wc1 = p['coord']['W1'].astype(jnp.bfloat16)
    bc1 = p['coord']['b1'][None, :]
    Wc2 = p['coord']['W2'].astype(jnp.bfloat16)
    bc2 = p['coord']['b2'][None, :]
    grid = (E_PAD // BE,)
    nreal = N_EDGES // BE - 1
    full = lambda r, c: pl.BlockSpec((r, c), lambda i: (0, 0))
    blk = lambda c: pl.BlockSpec((BE, c), lambda i: (i, 0))
    efspec = pl.BlockSpec((BE, DE), lambda i: (jnp.minimum(i, nreal), 0))
    return pl.pallas_call(
        _edge_body,
        grid=grid,
        in_specs=[blk(H), blk(H), efspec, full(1, H), full(DE, H),
                  full(1, H), full(H, H), full(1, H), full(H, H), full(1, H),
                  full(H, 1), full(1, 1)],
        out_specs=[blk(H), blk(H)],
        out_shape=[jax.ShapeDtypeStruct((E_PAD, H), jnp.float32),
                   jax.ShapeDtypeStruct((E_PAD, H), jnp.float32)],
    )(pre1, rij, edge_fea, w1s, W1e, b1, W2, b2, Wc1, bc1, Wc2, bc2)


def _node_upd_body(h_ref, equ_ref, gate_ref, aggm_ref, aggt_ref,
                   wequ_ref, Wn1h_ref, Wn1m_ref, bn1_ref, Wn2_ref, bn2_ref,
                   h_out_ref, equ_out_ref):
    msum = jnp.sum(aggm_ref[...], axis=0)
    tail = jnp.sum(aggt_ref[...], axis=0)
    cnt = tail[:, 3:4]
    fmean = tail / jnp.maximum(cnt, 1.0)
    mask = (jax.lax.broadcasted_iota(jnp.int32, (1, EQW), 1) < 3
            ).astype(jnp.float32)
    equ_out_ref[...] = (equ_ref[...] * wequ_ref[0, 0] * gate_ref[...]
                        + fmean) * mask
    h = h_ref[...]
    u = _silu(jnp.dot(h, Wn1h_ref[...], preferred_element_type=jnp.float32)
              + jnp.dot(msum, Wn1m_ref[...], preferred_element_type=jnp.float32)
              + bn1_ref[...])
    h_out_ref[...] = (jnp.dot(u, Wn2_ref[...], preferred_element_type=jnp.float32)
                      + bn2_ref[...] + h)


def _node_update(h, equ, gate, aggm, aggt, p):
    Wn1h = p['node']['W1'][:H]
    Wn1m = p['node']['W1'][H:]
    bn1 = p['node']['b1'][None, :]
    Wn2 = p['node']['W2']
    bn2 = p['node']['b2'][None, :]
    S = aggm.shape[0]
    grid = (N_NODES // BN,)
    full = lambda r, c: pl.BlockSpec((r, c), lambda i: (0, 0))
    blk = lambda c: pl.BlockSpec((BN, c), lambda i: (i, 0))
    return pl.pallas_call(
        _node_upd_body,
        grid=grid,
        in_specs=[blk(H), blk(EQW), blk(1),
                  pl.BlockSpec((S, BN, H), lambda i: (0, i, 0)),
                  pl.BlockSpec((S, BN, TAILW), lambda i: (0, i, 0)),
                  full(1, 1), full(H, H), full(H, H), full(1, H), full(H, H),
                  full(1, H)],
        out_specs=[blk(H), blk(EQW)],
        out_shape=[jax.ShapeDtypeStruct((N_NODES, H), jnp.float32),
                   jax.ShapeDtypeStruct((N_NODES, EQW), jnp.float32)],
    )(h, equ, gate, aggm, aggt, p['W_equ'], Wn1h, Wn1m, bn1, Wn2, bn2)


NW = 32
CH = 128
CHG = 64
E_PAD = NW * 80 * CH
TS = E_PAD // (NW * CH)
TG = E_PAD // (NW * CHG)
EPW = E_PAD // NW
NPS = N_NODES // 16


AW = H + EQW

TG0, TG1 = 190, 130
SC0, SC1 = 95, 65


def _sc_gather(A144, B144, row, col):
    mesh = plsc.VectorSubcoreMesh(core_axis_name="c", subcore_axis_name="s")

    @functools.partial(
        pl.kernel, mesh=mesh,
        compiler_params=pltpu.CompilerParams(use_tc_tiling_on_sc=False),
        out_type=[jax.ShapeDtypeStruct((E_PAD, H), jnp.float32),
                  jax.ShapeDtypeStruct((E_PAD, H), jnp.float32)],
        scratch_types=[pltpu.VMEM((TG0 * CHG,), jnp.int32),
                       pltpu.VMEM((TG0 * CHG,), jnp.int32),
                       pltpu.VMEM((CHG, AW), jnp.float32),
                       pltpu.VMEM((CHG, AW), jnp.float32),
                       pltpu.VMEM((CHG, AW), jnp.float32),
                       pltpu.VMEM((CHG, AW), jnp.float32),
                       pltpu.VMEM((CHG, H), jnp.float32),
                       pltpu.VMEM((CHG, H), jnp.float32),
                       pltpu.VMEM((CHG, EQW), jnp.float32),
                       pltpu.VMEM((CHG, EQW), jnp.float32),
                       pltpu.SemaphoreType.DMA,
                       pltpu.SemaphoreType.DMA,
                       pltpu.SemaphoreType.DMA,
                       pltpu.SemaphoreType.DMA],
    )
    def k(A_h, B_h, row_h, col_h, pre1_h, rij_h,
          rva, cva, av0, av1, bv0, bv1, p0, p1, r0, r1,
          sg0, sg1, sw0, sw1):
        c = lax.axis_index("c")
        s = lax.axis_index("s")
        tgc = TG0 - (TG0 - TG1) * c
        ebase = (c * (16 * TG0) + s * tgc) * CHG

        @pl.when(c == 0)
        def _():
            pltpu.sync_copy(row_h.at[pl.ds(ebase, TG0 * CHG)],
                            rva.at[pl.ds(0, TG0 * CHG)])
            pltpu.sync_copy(col_h.at[pl.ds(ebase, TG0 * CHG)],
                            cva.at[pl.ds(0, TG0 * CHG)])

        @pl.when(c == 1)
        def _():
            pltpu.sync_copy(row_h.at[pl.ds(ebase, TG1 * CHG)],
                            rva.at[pl.ds(0, TG1 * CHG)])
            pltpu.sync_copy(col_h.at[pl.ds(ebase, TG1 * CHG)],
                            cva.at[pl.ds(0, TG1 * CHG)])

        sets = ((av0, bv0, p0, r0, sg0, sw0),
                (av1, bv1, p1, r1, sg1, sw1))

        def g_copies(st, t):
            av, bv, _, _, sg, _ = st
            rs = rva.at[pl.ds(t * CHG, CHG)]
            cs = cva.at[pl.ds(t * CHG, CHG)]
            return (pltpu.make_async_copy(A_h.at[rs], av, sg),
                    pltpu.make_async_copy(B_h.at[cs], bv, sg))

        def w_copies(st, t):
            _, _, p, r, _, sw = st
            dst = ebase + t * CHG
            return (pltpu.make_async_copy(p, pre1_h.at[pl.ds(dst, CHG)], sw),
                    pltpu.make_async_copy(
                        r, rij_h.at[pl.ds(dst, CHG), pl.ds(0, EQW)], sw))

        def slot(si, t):
            st = sets[si]
            av, bv, p, r, _, _ = st
            for cp in g_copies(st, t):
                cp.wait()
            @pl.when(t >= 2)
            def _():
                for cp in w_copies(st, t - 2):
                    cp.wait()

            @pl.loop(0, CHG)
            def _(i):
                for j in range(0, H, 16):
                    p.at[i, pl.ds(j, 16)][...] = (
                        av.at[i, pl.ds(j, 16)][...]
                        + bv.at[i, pl.ds(j, 16)][...])
                r.at[i][...] = (av.at[i, pl.ds(H, EQW)][...]
                                - bv.at[i, pl.ds(H, EQW)][...])

            for cp in w_copies(st, t):
                cp.start()
            @pl.when(t + 2 < tgc)
            def _():
                for cp in g_copies(st, t + 2):
                    cp.start()

        for cp in g_copies(sets[0], 0):
            cp.start()
        for cp in g_copies(sets[1], 1):
            cp.start()

        @pl.loop(0, tgc, step=2)
        def _(t):
            slot(0, t)
            slot(1, t + 1)

        for cp in w_copies(sets[0], tgc - 2):
            cp.wait()
        for cp in w_copies(sets[1], tgc - 1):
            cp.wait()

    return k(A144, B144, row, col)


def _sc_scatter(m_out, tail_out, row):
    mesh = plsc.VectorSubcoreMesh(core_axis_name="c", subcore_axis_name="s")

    @functools.partial(
        pl.kernel, mesh=mesh,
        compiler_params=pltpu.CompilerParams(use_tc_tiling_on_sc=False),
        out_type=[jax.ShapeDtypeStruct((2, N_NODES, H), jnp.float32),
                  jax.ShapeDtypeStruct((2, N_NODES, TAILW), jnp.float32)],
        scratch_types=[pltpu.VMEM((CH,), jnp.int32),
                       pltpu.VMEM((CH, H), jnp.float32),
                       pltpu.VMEM((CH, TAILW), jnp.float32),
                       pltpu.VMEM_SHARED((N_NODES, H), jnp.float32),
                       pltpu.VMEM_SHARED((N_NODES, TAILW), jnp.float32)],
    )
    def k(m_h, t_h, row_h, outm_h, outt_h, idxv, mrows, trows, accm, acct):
        c = lax.axis_index("c")
        s = lax.axis_index("s")
        scc = SC0 - (SC0 - SC1) * c
        cbase = c * (16 * SC0) + s * scc

        @pl.loop(0, 125)
        def _(i):
            for j in range(0, H, 16):
                mrows.at[i, pl.ds(j, 16)][...] = jnp.zeros((16,), jnp.float32)
            trows.at[i][...] = jnp.zeros((16,), jnp.float32)

        @pl.loop(0, NPS, step=125)
        def _(r):
            pltpu.sync_copy(mrows.at[pl.ds(0, 125)],
                            accm.at[pl.ds(s * NPS + r, 125)])
            pltpu.sync_copy(trows.at[pl.ds(0, 125)],
                            acct.at[pl.ds(s * NPS + r, 125)])

        plsc.subcore_barrier()

        @pl.loop(cbase, cbase + scc)
        def _(chk):
            base = chk * CH
            pltpu.sync_copy(row_h.at[pl.ds(base, CH)], idxv)
            pltpu.sync_copy(m_h.at[pl.ds(base, CH)], mrows)
            pltpu.sync_copy(t_h.at[pl.ds(base, CH), pl.ds(0, TAILW)], trows)
            pltpu.sync_copy(mrows, accm.at[idxv], add=True)
            pltpu.sync_copy(trows, acct.at[idxv], add=True)

        plsc.subcore_barrier()
        pltpu.sync_copy(accm.at[pl.ds(s * NPS, NPS)],
                        outm_h.at[c].at[pl.ds(s * NPS, NPS)])
        pltpu.sync_copy(acct.at[pl.ds(s * NPS, NPS)],
                        outt_h.at[c].at[pl.ds(s * NPS, NPS)])

    return k(m_out, tail_out, row)


def kernel(x, h, edge_index, edge_fea, W_emb, b_emb, params):
    pad_e = E_PAD - N_EDGES
    row = jnp.pad(edge_index[0].astype(jnp.int32), (0, pad_e))
    col = jnp.pad(edge_index[1].astype(jnp.int32), (0, pad_e))
    h = _embed(h, W_emb, b_emb)
    equ = jnp.pad(x, ((0, 0), (0, EQW - 3)))
    for p in params:
        A, B, gate = _node_pre(h, p)
        A144 = jnp.concatenate([A, equ], axis=1)
        B144 = jnp.concatenate([B, equ], axis=1)
        pre1, rij = _sc_gather(A144, B144, row, col)
        m_out, tail_out = _edge_mlp(pre1, rij, edge_fea, p)
        aggm, aggt = _sc_scatter(m_out, tail_out, row)
        h, equ = _node_update(h, equ, gate, aggm, aggt, p)
    return equ[:, :3]
